# Initial kernel scaffold; baseline (speedup 1.0000x reference)
import jax
import jax.numpy as jnp
from jax.experimental import pallas as pl


def kernel(batch_rgn_sqn, encoding):
    raise NotImplementedError("write your pallas kernel here")


# speedup vs baseline: 1.2194x; 1.2194x over previous
import functools

import jax
import jax.numpy as jnp
from jax import lax
from jax.experimental import pallas as pl
from jax.experimental.pallas import tpu as pltpu
from jax.experimental.pallas import tpu_sc as plsc

_PS_DIM = 64
_TABLE_ROWS = 200
_LANES = 16


@functools.lru_cache(maxsize=None)
def _gather_call(total_rows, chunk):
    info = plsc.get_sparse_core_info()
    nw = info.num_cores * info.num_subcores
    per_w = total_rows // nw
    n_chunks = per_w // chunk
    assert per_w * nw == total_rows and n_chunks * chunk == per_w

    mesh = plsc.VectorSubcoreMesh(core_axis_name="c", subcore_axis_name="s")

    @functools.partial(
        pl.kernel,
        mesh=mesh,
        compiler_params=pltpu.CompilerParams(needs_layout_passes=False),
        out_type=jax.ShapeDtypeStruct((total_rows * _PS_DIM,), jnp.float32),
        scratch_types=[
            pltpu.VMEM((_TABLE_ROWS * _PS_DIM,), jnp.float32),
            pltpu.VMEM((chunk,), jnp.int32),
            pltpu.VMEM((chunk * _PS_DIM,), jnp.float32),
        ],
    )
    def k(table_hbm, idx_hbm, out_hbm, table_v, idx_v, rows_v):
        wid = lax.axis_index("s") * info.num_cores + lax.axis_index("c")
        base = wid * per_w
        pltpu.sync_copy(table_hbm, table_v)
        lane = lax.iota(jnp.int32, _LANES)
        out_lane = lane * _PS_DIM

        def chunk_body(g, carry):
            row0 = base + g * chunk
            pltpu.sync_copy(idx_hbm.at[pl.ds(row0, chunk)], idx_v)

            def j_body(j, c):
                idxv = idx_v[pl.ds(j * _LANES, _LANES)]
                fb = idxv * _PS_DIM
                op = out_lane + j * (_LANES * _PS_DIM)
                for d in range(_PS_DIM):
                    v = plsc.load_gather(table_v, [fb])
                    plsc.store_scatter(rows_v, [op], v)
                    if d != _PS_DIM - 1:
                        fb = fb + 1
                        op = op + 1
                return c

            lax.fori_loop(0, chunk // _LANES, j_body, 0, unroll=False)
            pltpu.sync_copy(
                rows_v, out_hbm.at[pl.ds(row0 * _PS_DIM, chunk * _PS_DIM)]
            )
            return carry

        lax.fori_loop(0, n_chunks, chunk_body, 0, unroll=False)

    return k


def kernel(batch_rgn_sqn, encoding):
    b, l = batch_rgn_sqn.shape
    table = encoding[:_TABLE_ROWS, :_PS_DIM].reshape(-1)
    idx = batch_rgn_sqn.reshape(-1).astype(jnp.int32)
    out = _gather_call(b * l, 1024)(table, idx)
    return out.reshape(b, l, _PS_DIM)


# speedup vs baseline: 3.1083x; 2.5491x over previous
import functools

import jax
import jax.numpy as jnp
from jax import lax
from jax.experimental import pallas as pl
from jax.experimental.pallas import tpu as pltpu
from jax.experimental.pallas import tpu_sc as plsc

_PS_DIM = 64
_TABLE_ROWS = 200
_LANES = 16


@functools.lru_cache(maxsize=None)
def _gather_call(total_rows, chunk):
    info = plsc.get_sparse_core_info()
    nw = info.num_cores * info.num_subcores
    per_w = total_rows // nw
    n_chunks = per_w // chunk
    assert per_w * nw == total_rows and n_chunks * chunk == per_w

    mesh = plsc.VectorSubcoreMesh(core_axis_name="c", subcore_axis_name="s")

    @functools.partial(
        pl.kernel,
        mesh=mesh,
        compiler_params=pltpu.CompilerParams(needs_layout_passes=False),
        out_type=jax.ShapeDtypeStruct((total_rows * _PS_DIM,), jnp.float32),
        scratch_types=[
            pltpu.VMEM((_TABLE_ROWS * _PS_DIM,), jnp.float32),
            pltpu.VMEM((chunk,), jnp.int32),
            pltpu.VMEM((chunk * _PS_DIM,), jnp.float32),
        ],
    )
    def k(table_hbm, idx_hbm, out_hbm, table_v, idx_v, rows_v):
        wid = lax.axis_index("s") * info.num_cores + lax.axis_index("c")
        base = wid * per_w
        pltpu.sync_copy(table_hbm, table_v)
        unroll = _LANES
        nq = _PS_DIM // _LANES

        def chunk_body(g, carry):
            row0 = base + g * chunk
            pltpu.sync_copy(idx_hbm.at[pl.ds(row0, chunk)], idx_v)

            def j_body(j, c):
                r0 = j * unroll
                idxv = idx_v[pl.ds(r0, _LANES)] * _PS_DIM
                for u in range(unroll):
                    off = idxv[u]
                    ob = (r0 + u) * _PS_DIM
                    for q in range(nq):
                        rows_v[pl.ds(ob + q * _LANES, _LANES)] = table_v[
                            pl.ds(off + q * _LANES, _LANES)
                        ]
                return c

            lax.fori_loop(0, chunk // unroll, j_body, 0, unroll=False)
            pltpu.sync_copy(
                rows_v, out_hbm.at[pl.ds(row0 * _PS_DIM, chunk * _PS_DIM)]
            )
            return carry

        lax.fori_loop(0, n_chunks, chunk_body, 0, unroll=False)

    return k


def kernel(batch_rgn_sqn, encoding):
    b, l = batch_rgn_sqn.shape
    table = encoding[:_TABLE_ROWS, :_PS_DIM].reshape(-1)
    idx = batch_rgn_sqn.reshape(-1).astype(jnp.int32)
    out = _gather_call(b * l, 1024)(table, idx)
    return out.reshape(b, l, _PS_DIM)
